# baseline (speedup 1.0000x reference)
import functools

import jax
import jax.numpy as jnp
from jax import lax
from jax.experimental import pallas as pl
from jax.experimental.pallas import tpu as pltpu
from jax.experimental.pallas import tpu_sc as plsc

NC = 2
NS = 16
NW = NC * NS
L = 16
BB = 128
HG = 8
NBUF = 2
NG = 4


def _emb_call(H, B, D, n_hg):
    mesh = plsc.VectorSubcoreMesh(core_axis_name="c", subcore_axis_name="s")
    n_grp = BB // L
    RW = 2 * D

    @functools.partial(
        pl.kernel,
        mesh=mesh,
        out_type=jax.ShapeDtypeStruct((H, D, B), jnp.float32),
        scratch_types=[
            pltpu.VMEM((HG, BB), jnp.int32),
            pltpu.VMEM((HG, BB), jnp.int32),
            pltpu.VMEM((HG, BB), jnp.int32),
            [pltpu.VMEM((BB, RW), jnp.float32) for _ in range(NG)],
            [pltpu.VMEM((D, BB), jnp.float32) for _ in range(NBUF)],
            [pltpu.SemaphoreType.DMA for _ in range(NG)],
            [pltpu.SemaphoreType.DMA for _ in range(NBUF)],
        ],
        compiler_params=pltpu.CompilerParams(
            use_tc_tiling_on_sc=True, needs_layout_passes=False),
    )
    def emb(xt_hbm, tab_hbm, out_hbm, idx_v, half_v, base_v, rows_v, slab_v,
            sem_g, sem_o):
        wid = lax.axis_index("s") * NC + lax.axis_index("c")
        b0 = wid * BB
        lane = lax.broadcasted_iota(jnp.int32, (L,), 0)

        def gather_copy(r, k):
            return pltpu.make_async_copy(
                tab_hbm.at[half_v.at[r]], rows_v[k], sem_g[k])

        def slab_copy(h, k):
            return pltpu.make_async_copy(
                slab_v[k],
                out_hbm.at[h, :, pl.ds(b0, BB)],
                sem_o[k],
            )

        def prep(g):
            pltpu.sync_copy(xt_hbm.at[pl.ds(g * HG, HG), pl.ds(b0, BB)],
                            idx_v)
            for r in range(HG):
                for c in range(n_grp):
                    s = pl.ds(c * L, L)
                    v = idx_v[r, s]
                    half_v[r, s] = lax.shift_right_logical(v, 1)
                    base_v[r, s] = (lane + c * L) * RW + (v & 1) * D

        zerov = lax.broadcasted_iota(jnp.int32, (L,), 0) * 0

        def transpose_h(r, kg, ks):
            cols0 = tuple(
                base_v[r, pl.ds(c * L, L)] for c in range(n_grp))

            @plsc.parallel_loop(0, D, 1, unroll=8)
            def _(d):
                for c in range(n_grp):
                    vals = plsc.load_gather(rows_v[kg], [zerov, cols0[c] + d])
                    slab_v[ks][d, pl.ds(c * L, L)] = vals

        def unit(g, carry):
            prep(g)
            for r in range(NG):
                gather_copy(r, r).start()
            for r in range(HG):
                h = g * HG + r
                kg = r % NG
                ks = r % NBUF
                @pl.when((g > 0) | (r >= NBUF))
                def _():
                    slab_copy(h - NBUF, ks).wait()
                gather_copy(r, kg).wait()
                transpose_h(r, kg, ks)
                slab_copy(h, ks).start()
                if r + NG < HG:
                    gather_copy(r + NG, kg).start()
            return carry

        lax.fori_loop(0, n_hg, unit, 0)
        for r in range(NBUF):
            slab_copy(H - NBUF + r, (H - NBUF + r) % NBUF).wait()

    return emb


def kernel(x, table):
    B, H = x.shape
    V, D = table.shape
    xt = x.T.astype(jnp.int32)
    tab2 = jnp.concatenate([table[0::2, :], table[1::2, :]], axis=1)
    n_hg = H // HG
    q = _emb_call(H, B, D, n_hg)(xt, tab2)
    return q.transpose(2, 0, 1)


# speedup vs baseline: 5.9110x; 5.9110x over previous
import functools

import jax
import jax.numpy as jnp
from jax import lax
from jax.experimental import pallas as pl
from jax.experimental.pallas import tpu as pltpu
from jax.experimental.pallas import tpu_sc as plsc

NC = 2
NS = 16
NW = NC * NS
L = 16
BB = 128
HG = 8
NBUF = 2
NG = 4


def _emb_call(H, B, D, n_hg):
    mesh = plsc.VectorSubcoreMesh(core_axis_name="c", subcore_axis_name="s")
    n_grp = BB // L
    RW = 2 * D

    @functools.partial(
        pl.kernel,
        mesh=mesh,
        out_type=jax.ShapeDtypeStruct((H, D, B), jnp.float32),
        scratch_types=[
            pltpu.VMEM((HG, BB), jnp.int32),
            pltpu.VMEM((HG, BB), jnp.int32),
            pltpu.VMEM((HG, BB), jnp.int32),
            [pltpu.VMEM((BB, RW), jnp.float32) for _ in range(NG)],
            [pltpu.VMEM((D, BB), jnp.float32) for _ in range(NBUF)],
            [pltpu.SemaphoreType.DMA for _ in range(NG)],
            [pltpu.SemaphoreType.DMA for _ in range(NBUF)],
        ],
        compiler_params=pltpu.CompilerParams(
            use_tc_tiling_on_sc=True, needs_layout_passes=False),
    )
    def emb(xt_hbm, tab_hbm, out_hbm, idx_v, half_v, base_v, rows_v, slab_v,
            sem_g, sem_o):
        wid = lax.axis_index("s") * NC + lax.axis_index("c")
        b0 = wid * BB
        lane = lax.broadcasted_iota(jnp.int32, (L,), 0)

        def gather_copy(r, k):
            return pltpu.make_async_copy(
                tab_hbm.at[half_v.at[r]], rows_v[k], sem_g[k])

        def slab_copy(h, k):
            return pltpu.make_async_copy(
                slab_v[k],
                out_hbm.at[h, :, pl.ds(b0, BB)],
                sem_o[k],
            )

        def prep(g):
            pltpu.sync_copy(xt_hbm.at[pl.ds(g * HG, HG), pl.ds(b0, BB)],
                            idx_v)
            for r in range(HG):
                for c in range(n_grp):
                    s = pl.ds(c * L, L)
                    v = idx_v[r, s]
                    half_v[r, s] = lax.shift_right_logical(v, 1)
                    base_v[r, s] = (lane + c * L) * RW + (v & 1) * D

        zerov = lax.broadcasted_iota(jnp.int32, (L,), 0) * 0

        def transpose_h(r, kg, ks):
            cols0 = tuple(
                base_v[r, pl.ds(c * L, L)] for c in range(n_grp))

            @plsc.parallel_loop(0, D, 1, unroll=2)
            def _(d):
                vals = [
                    plsc.load_gather(rows_v[kg], [zerov, cols0[c] + d])
                    for c in range(n_grp)
                ]
                for c in range(n_grp):
                    slab_v[ks][d, pl.ds(c * L, L)] = vals[c]

        def unit(g, carry):
            prep(g)
            for r in range(NG):
                gather_copy(r, r).start()
            for r in range(HG):
                h = g * HG + r
                kg = r % NG
                ks = r % NBUF
                @pl.when((g > 0) | (r >= NBUF))
                def _():
                    slab_copy(h - NBUF, ks).wait()
                gather_copy(r, kg).wait()
                transpose_h(r, kg, ks)
                slab_copy(h, ks).start()
                if r + NG < HG:
                    gather_copy(r + NG, kg).start()
            return carry

        lax.fori_loop(0, n_hg, unit, 0)
        for r in range(NBUF):
            slab_copy(H - NBUF + r, (H - NBUF + r) % NBUF).wait()

    return emb


def kernel(x, table):
    B, H = x.shape
    V, D = table.shape
    xt = x.T.astype(jnp.int32)
    tab2 = table.reshape(V // 2, 2 * D)
    n_hg = H // HG
    q = _emb_call(H, B, D, n_hg)(xt, tab2)
    return q.transpose(2, 0, 1)


# speedup vs baseline: 8.2051x; 1.3881x over previous
import functools

import jax
import jax.numpy as jnp
from jax import lax
from jax.experimental import pallas as pl
from jax.experimental.pallas import tpu as pltpu
from jax.experimental.pallas import tpu_sc as plsc

NC = 2
NS = 16
NW = NC * NS
C = 400
NBUF = 2


def _emb_call(c_per_w, n_steps, D, N):
    mesh = plsc.VectorSubcoreMesh(core_axis_name="c", subcore_axis_name="s")
    RW = 2 * D

    @functools.partial(
        pl.kernel,
        mesh=mesh,
        out_type=jax.ShapeDtypeStruct((N, 2 * D), jnp.float32),
        scratch_types=[
            [pltpu.VMEM((C,), jnp.int32) for _ in range(NBUF)],
            [pltpu.VMEM((C, RW), jnp.float32) for _ in range(NBUF)],
            [pltpu.SemaphoreType.DMA for _ in range(NBUF)],
            [pltpu.SemaphoreType.DMA for _ in range(NBUF)],
        ],
        compiler_params=pltpu.CompilerParams(
            use_tc_tiling_on_sc=True, needs_layout_passes=False),
    )
    def emb(idx_hbm, tab_hbm, out_hbm, idx_v, rows_v, sem_g, sem_o):
        wid = lax.axis_index("s") * NC + lax.axis_index("c")
        r0 = wid * c_per_w

        def gather_copy(i, b):
            return pltpu.make_async_copy(
                tab_hbm.at[idx_v[b]], rows_v[b], sem_g[b])

        def out_copy(i, b):
            return pltpu.make_async_copy(
                rows_v[b],
                out_hbm.at[pl.ds(r0 + i * C, C), :],
                sem_o[b],
            )

        def fire(i, b, wait_out):
            pltpu.sync_copy(idx_hbm.at[pl.ds(r0 + i * C, C)], idx_v[b])
            if wait_out:
                out_copy(i - NBUF, b).wait()
            gather_copy(i, b).start()

        def retire(i, b):
            gather_copy(i, b).wait()
            out_copy(i, b).start()

        fire(0, 0, False)
        fire(1, 1, False)
        retire(0, 0)

        def steady(o, carry):
            for k in range(NBUF):
                i = o * NBUF + k
                fire(i, k, True)
                retire(i - 1, (k - 1) % NBUF)
            return carry

        lax.fori_loop(1, n_steps // NBUF, steady, 0)
        retire(n_steps - 1, (n_steps - 1) % NBUF)
        for i in range(n_steps - NBUF, n_steps):
            out_copy(i, i % NBUF).wait()

    return emb


def kernel(x, table):
    B, H = x.shape
    V, D = table.shape
    N = B * H
    c_per_w = N // NW
    n_steps = c_per_w // C
    idx_flat = x.reshape(N).astype(jnp.int32)
    tab2 = jnp.broadcast_to(table[:, None, :], (V, 2, D)).reshape(V, 2 * D)
    out2 = _emb_call(c_per_w, n_steps, D, N)(idx_flat, tab2)
    return out2[:, :D].reshape(B, H, D)


# speedup vs baseline: 8.2350x; 1.0036x over previous
import functools

import jax
import jax.numpy as jnp
from jax import lax
from jax.experimental import pallas as pl
from jax.experimental.pallas import tpu as pltpu
from jax.experimental.pallas import tpu_sc as plsc

NC = 2
NS = 16
NW = NC * NS
C = 400
NBUF = 2


def _emb_call(c_per_w, n_steps, D, N):
    mesh = plsc.VectorSubcoreMesh(core_axis_name="c", subcore_axis_name="s")
    RW = 2 * D

    @functools.partial(
        pl.kernel,
        mesh=mesh,
        out_type=jax.ShapeDtypeStruct((N, 2 * D), jnp.float32),
        scratch_types=[
            [pltpu.VMEM((C,), jnp.int32) for _ in range(NBUF)],
            [pltpu.VMEM((C, RW), jnp.float32) for _ in range(NBUF)],
            [pltpu.SemaphoreType.DMA for _ in range(NBUF)],
            [pltpu.SemaphoreType.DMA for _ in range(NBUF)],
        ],
        compiler_params=pltpu.CompilerParams(
            use_tc_tiling_on_sc=True, needs_layout_passes=False),
    )
    def emb(idx_hbm, tab_hbm, out_hbm, idx_v, rows_v, sem_g, sem_o):
        wid = lax.axis_index("s") * NC + lax.axis_index("c")
        r0 = wid * c_per_w

        def gather_copy(i, b):
            return pltpu.make_async_copy(
                tab_hbm.at[idx_v[b]], rows_v[b], sem_g[b])

        def out_copy(i, b):
            return pltpu.make_async_copy(
                rows_v[b],
                out_hbm.at[pl.ds(r0 + i * C, C), :],
                sem_o[b],
            )

        def fire(i, b, wait_out):
            pltpu.sync_copy(idx_hbm.at[pl.ds(r0 + i * C, C)], idx_v[b])
            if wait_out:
                out_copy(i - NBUF, b).wait()
            gather_copy(i, b).start()

        def retire(i, b):
            gather_copy(i, b).wait()
            out_copy(i, b).start()

        fire(0, 0, False)
        fire(1, 1, False)
        retire(0, 0)

        def steady(o, carry):
            for k in range(NBUF):
                i = o * NBUF + k
                fire(i, k, True)
                retire(i - 1, (k - 1) % NBUF)
            return carry

        lax.fori_loop(1, n_steps // NBUF, steady, 0)
        retire(n_steps - 1, (n_steps - 1) % NBUF)
        for i in range(n_steps - NBUF, n_steps):
            out_copy(i, i % NBUF).wait()

    return emb


def kernel(x, table):
    B, H = x.shape
    V, D = table.shape
    N = B * H
    c_per_w = N // NW
    n_steps = c_per_w // C
    idx_flat = x.reshape(N).astype(jnp.int32)
    tab2 = jnp.concatenate([table, table], axis=1)
    out2 = _emb_call(c_per_w, n_steps, D, N)(idx_flat, tab2)
    return out2[:, :D].reshape(B, H, D)
